# baseline (speedup 1.0000x reference)
import jax
import jax.numpy as jnp
from jax import lax
from jax.experimental import pallas as pl
from jax.experimental.pallas import tpu as pltpu
from jax.experimental.pallas import tpu_sc as plsc

_H = 32
_W = 32
_B = 16
_D = 256


def _pos_emb_body(row_hbm, col_hbm, out_hbm, slab, bsem, osem):
    s = lax.axis_index("s")
    for half in range(2):
        i = s + half * 16
        build = [
            pltpu.async_copy(row_hbm.at[i], slab.at[half, 0, pl.ds(0, _D)], bsem),
            pltpu.async_copy(
                col_hbm.at[pl.ds(0, _W)], slab.at[half, :, pl.ds(_D, _D)], bsem
            ),
        ]
        for c in build:
            c.wait()
        vregs = [slab[half, 0, pl.ds(k * 16, 16)] for k in range(_D // 16)]
        for j in range(1, _W):
            for k in range(_D // 16):
                slab[half, j, pl.ds(k * 16, 16)] = vregs[k]
    copies = []
    for half in range(2):
        i = s + half * 16
        copies += [
            pltpu.async_copy(slab.at[half], out_hbm.at[b, i], osem)
            for b in range(_B)
        ]
    for c in copies:
        c.wait()


def kernel(tensor, row_embed, col_embed):
    del tensor
    mesh = plsc.VectorSubcoreMesh(
        core_axis_name="c", subcore_axis_name="s", num_cores=1
    )
    f = pl.kernel(
        _pos_emb_body,
        out_type=jax.ShapeDtypeStruct((_B, _H, _W, 2 * _D), jnp.float32),
        mesh=mesh,
        scratch_types=[
            pltpu.VMEM((2, _W, 2 * _D), jnp.float32),
            pltpu.SemaphoreType.DMA,
            pltpu.SemaphoreType.DMA,
        ],
    )
    return f(row_embed, col_embed)


# speedup vs baseline: 1.3059x; 1.3059x over previous
import jax
import jax.numpy as jnp
from jax import lax
from jax.experimental import pallas as pl
from jax.experimental.pallas import tpu as pltpu
from jax.experimental.pallas import tpu_sc as plsc

_H = 32
_W = 32
_B = 16
_D = 256


def _pos_emb_body(row_hbm, col_hbm, out_hbm, slab, bsem, osem):
    i = lax.axis_index("s") * 2 + lax.axis_index("c")
    build = [
        pltpu.async_copy(row_hbm.at[i], slab.at[0, pl.ds(0, _D)], bsem),
        pltpu.async_copy(col_hbm.at[pl.ds(0, _W)], slab.at[:, pl.ds(_D, _D)], bsem),
    ]
    for c in build:
        c.wait()
    vregs = [slab[0, pl.ds(k * 16, 16)] for k in range(_D // 16)]
    for j in range(1, _W):
        for k in range(_D // 16):
            slab[j, pl.ds(k * 16, 16)] = vregs[k]
    copies = [pltpu.async_copy(slab, out_hbm.at[b, i], osem) for b in range(_B)]
    for c in copies:
        c.wait()


def kernel(tensor, row_embed, col_embed):
    del tensor
    mesh = plsc.VectorSubcoreMesh(core_axis_name="c", subcore_axis_name="s")
    f = pl.kernel(
        _pos_emb_body,
        out_type=jax.ShapeDtypeStruct((_B, _H, _W, 2 * _D), jnp.float32),
        mesh=mesh,
        scratch_types=[
            pltpu.VMEM((_W, 2 * _D), jnp.float32),
            pltpu.SemaphoreType.DMA,
            pltpu.SemaphoreType.DMA,
        ],
    )
    return f(row_embed, col_embed)
